# baseline (speedup 1.0000x reference)
import functools

import jax
import jax.numpy as jnp
from jax import lax
from jax.experimental import pallas as pl
from jax.experimental.pallas import tpu as pltpu
from jax.experimental.pallas import tpu_sc as plsc

B, N, C, D = 4, 50000, 18, 128
NPAD = 50176
R = NPAD // 128
K_CROSS = 1024
K_SEL = 256
NEG = -3.0e38
BIGI = 1 << 30


def _tc_topk_body(cls_ref, cen_ref, pts_ref, ocross_ref, osort_ref, opts_ref, s_ref, m_ref):
    b = pl.program_id(0)
    b_off = b * N

    m = cls_ref[0, 0]
    for c in range(1, C):
        m = jnp.maximum(m, cls_ref[0, c])
    s = jax.nn.sigmoid(m) * jax.nn.sigmoid(cen_ref[0])
    riota2 = lax.broadcasted_iota(jnp.int32, (R, 128), 0)
    liota2 = lax.broadcasted_iota(jnp.int32, (R, 128), 1)
    gidx2 = riota2 * 128 + liota2
    s = jnp.where(gidx2 < N, s, jnp.float32(-1.0))
    s_ref[...] = s
    m_ref[...] = jnp.max(s, axis=1, keepdims=True)

    riota = lax.broadcasted_iota(jnp.int32, (R, 1), 0)
    liota = lax.broadcasted_iota(jnp.int32, (1, 128), 1)
    kiota = lax.broadcasted_iota(jnp.int32, (1, 1, K_CROSS), 2)

    def body(k, carry):
        mv = m_ref[...]
        maxv = jnp.max(mv)
        r = jnp.min(jnp.where(mv == maxv, riota, BIGI))
        row = s_ref[pl.ds(r, 1), :]
        l = jnp.min(jnp.where(row == maxv, liota, BIGI))
        gidx = r * 128 + l + b_off
        ocross_ref[...] = jnp.where(kiota == k, gidx, ocross_ref[...])
        newrow = jnp.where(liota == l, NEG, row)
        s_ref[pl.ds(r, 1), :] = newrow
        m_ref[pl.ds(r, 1), :] = jnp.max(newrow, axis=1, keepdims=True)
        return carry

    lax.fori_loop(0, K_CROSS, body, jnp.int32(0))

    piota = lax.broadcasted_iota(jnp.int32, (1, 1, K_SEL), 2)
    piota3 = lax.broadcasted_iota(jnp.int32, (1, 3, K_SEL), 2)
    liota3 = lax.broadcasted_iota(jnp.int32, (3, 128), 1)
    arr0 = ocross_ref[:, :, pl.ds(0, K_SEL)] - b_off

    def sbody(p, arr):
        mn = jnp.min(arr)
        osort_ref[...] = jnp.where(piota == p, mn, osort_ref[...])
        r = mn // 128
        l = mn % 128
        blk = jnp.squeeze(pts_ref[0, :, pl.ds(r, 1), :], axis=1)
        vals = jnp.sum(jnp.where(liota3 == l, blk, 0.0), axis=1, keepdims=True)
        opts_ref[...] = jnp.where(piota3 == p, vals[None], opts_ref[...])
        return jnp.where(arr == mn, BIGI, arr)

    lax.fori_loop(0, K_SEL, sbody, arr0)


_tc_topk = pl.pallas_call(
    _tc_topk_body,
    grid=(B,),
    in_specs=[
        pl.BlockSpec((1, C, R, 128), lambda b: (b, 0, 0, 0)),
        pl.BlockSpec((1, R, 128), lambda b: (b, 0, 0)),
        pl.BlockSpec((1, 3, R, 128), lambda b: (b, 0, 0, 0)),
    ],
    out_specs=[
        pl.BlockSpec((1, 1, K_CROSS), lambda b: (b, 0, 0)),
        pl.BlockSpec((1, 1, K_SEL), lambda b: (b, 0, 0)),
        pl.BlockSpec((1, 3, K_SEL), lambda b: (b, 0, 0)),
    ],
    out_shape=[
        jax.ShapeDtypeStruct((B, 1, K_CROSS), jnp.int32),
        jax.ShapeDtypeStruct((B, 1, K_SEL), jnp.int32),
        jax.ShapeDtypeStruct((B, 3, K_SEL), jnp.float32),
    ],
    scratch_shapes=[
        pltpu.VMEM((R, 128), jnp.float32),
        pltpu.VMEM((R, 1), jnp.float32),
    ],
)

N_FEAT_ROWS = B * K_CROSS + B * K_SEL
NW = 32
F_PER_W = N_FEAT_ROWS // NW


@functools.partial(
    pl.kernel,
    out_type=jax.ShapeDtypeStruct((N_FEAT_ROWS, D), jnp.float32),
    mesh=plsc.VectorSubcoreMesh(core_axis_name="c", subcore_axis_name="s"),
    scratch_types=[
        pltpu.VMEM((2, 80), jnp.int32),
        pltpu.VMEM((F_PER_W, D), jnp.float32),
        pltpu.SemaphoreType.DMA,
    ],
)
def _sc_gather(feat_hbm, fidx_hbm, ofeat, fidx_v, frows_v, semf):
    wid = lax.axis_index("s") * 2 + lax.axis_index("c")
    pltpu.sync_copy(fidx_hbm.at[pl.ds(2 * wid, 2)], fidx_v)
    c1 = pltpu.async_copy(feat_hbm.at[fidx_v.at[0]], frows_v.at[pl.ds(0, 80)], semf)
    c2 = pltpu.async_copy(feat_hbm.at[fidx_v.at[1]], frows_v.at[pl.ds(80, 80)], semf)
    c1.wait()
    c2.wait()
    pltpu.sync_copy(frows_v, ofeat.at[pl.ds(F_PER_W * wid, F_PER_W)])


def kernel(centerness, cls_scores, points, features):
    cls_t = jnp.transpose(cls_scores, (0, 2, 1))
    cls_t = jnp.pad(cls_t, ((0, 0), (0, 0), (0, NPAD - N))).reshape(B, C, R, 128)
    cen = jnp.pad(centerness.reshape(B, N), ((0, 0), (0, NPAD - N))).reshape(B, R, 128)

    pts_t = jnp.pad(jnp.transpose(points, (0, 2, 1)),
                    ((0, 0), (0, 0), (0, NPAD - N))).reshape(B, 3, R, 128)

    cross_g, sort3, sel_pts_t = _tc_topk(cls_t, cen, pts_t)
    sort_inds = sort3.reshape(B, K_SEL)

    offs = (jnp.arange(B, dtype=jnp.int32) * N).reshape(B, 1)
    sel_g = (sort_inds + offs).reshape(-1)
    fidx = jnp.concatenate([cross_g.reshape(-1), sel_g]).reshape(64, 80)

    feat_flat = features.reshape(B * N, D)
    gfeat = _sc_gather(feat_flat, fidx)

    cross_features = gfeat[: B * K_CROSS].reshape(B, K_CROSS, D)
    sel_features = gfeat[B * K_CROSS:].reshape(B, K_SEL, D)
    sel_points = jnp.transpose(sel_pts_t, (0, 2, 1))
    return (sel_points, sel_features, sort_inds, cross_features)


# speedup vs baseline: 1.3002x; 1.3002x over previous
import functools

import jax
import jax.numpy as jnp
from jax import lax
from jax.experimental import pallas as pl
from jax.experimental.pallas import tpu as pltpu
from jax.experimental.pallas import tpu_sc as plsc

B, N, C, D = 4, 50000, 18, 128
NPAD = 50176
R = NPAD // 128
K_CROSS = 1024
K_SEL = 256
NEG = -3.0e38
BIGI = 1 << 30


G = R // 8


def _tc_topk_body(cls_ref, cen_ref, pts_ref, ocross_ref, osort_ref, opts_ref, s_ref, m_ref):
    riota2 = lax.broadcasted_iota(jnp.int32, (R, 128), 0)
    liota2 = lax.broadcasted_iota(jnp.int32, (R, 128), 1)
    gidx2 = riota2 * 128 + liota2
    giota = lax.broadcasted_iota(jnp.int32, (G, 128), 0)
    riota8 = lax.broadcasted_iota(jnp.int32, (8, 128), 0)
    liota8 = lax.broadcasted_iota(jnp.int32, (8, 128), 1)
    liota1 = lax.broadcasted_iota(jnp.int32, (1, 128), 1)
    piota3 = lax.broadcasted_iota(jnp.int32, (3, K_SEL), 1)
    liota3 = lax.broadcasted_iota(jnp.int32, (3, 128), 1)

    for b in range(B):
        m = cls_ref[b, 0]
        for c in range(1, C):
            m = jnp.maximum(m, cls_ref[b, c])
        s = jax.nn.sigmoid(m) * jax.nn.sigmoid(cen_ref[b])
        s = jnp.where(gidx2 < N, s, jnp.float32(-1.0))
        s_ref[b] = s
        m_ref[b] = jnp.max(s.reshape(G, 8, 128), axis=1)

    def body(k, carry):
        kr = k // 128
        kl = k % 128
        for b in range(B):
            m1 = m_ref[b]
            maxv = jnp.max(m1)
            g = jnp.min(jnp.where(m1 == maxv, giota, BIGI))
            s8 = s_ref[b, pl.ds(g * 8, 8), :]
            gidx8 = (riota8 + g * 8) * 128 + liota8
            amin = jnp.min(jnp.where(s8 == maxv, gidx8, BIGI))
            row = ocross_ref[b, pl.ds(kr, 1), :]
            ocross_ref[b, pl.ds(kr, 1), :] = jnp.where(liota1 == kl, amin + b * N, row)
            s8n = jnp.where(gidx8 == amin, NEG, s8)
            s_ref[b, pl.ds(g * 8, 8), :] = s8n
            m_ref[b, pl.ds(g, 1), :] = jnp.max(s8n, axis=0, keepdims=True)
        return carry

    lax.fori_loop(0, K_CROSS, body, jnp.int32(0))

    def sbody(p, arrs):
        pr = p // 128
        pl_ = p % 128
        out = []
        for b in range(B):
            arr = arrs[b]
            mn = jnp.min(arr)
            row = osort_ref[b, pl.ds(pr, 1), :]
            osort_ref[b, pl.ds(pr, 1), :] = jnp.where(liota1 == pl_, mn, row)
            r = mn // 128
            l = mn % 128
            blk = jnp.squeeze(pts_ref[b, :, pl.ds(r, 1), :], axis=1)
            vals = jnp.sum(jnp.where(liota3 == l, blk, 0.0), axis=1, keepdims=True)
            opts_ref[b] = jnp.where(piota3 == p, vals, opts_ref[b])
            out.append(jnp.where(arr == mn, BIGI, arr))
        return tuple(out)

    arr0 = tuple(ocross_ref[b, pl.ds(0, K_SEL // 128), :] - b * N for b in range(B))
    lax.fori_loop(0, K_SEL, sbody, arr0)


_tc_topk = pl.pallas_call(
    _tc_topk_body,
    grid=(1,),
    in_specs=[
        pl.BlockSpec((B, C, R, 128), lambda i: (0, 0, 0, 0)),
        pl.BlockSpec((B, R, 128), lambda i: (0, 0, 0)),
        pl.BlockSpec((B, 3, R, 128), lambda i: (0, 0, 0, 0)),
    ],
    out_specs=[
        pl.BlockSpec((B, K_CROSS // 128, 128), lambda i: (0, 0, 0)),
        pl.BlockSpec((B, K_SEL // 128, 128), lambda i: (0, 0, 0)),
        pl.BlockSpec((B, 3, K_SEL), lambda i: (0, 0, 0)),
    ],
    out_shape=[
        jax.ShapeDtypeStruct((B, K_CROSS // 128, 128), jnp.int32),
        jax.ShapeDtypeStruct((B, K_SEL // 128, 128), jnp.int32),
        jax.ShapeDtypeStruct((B, 3, K_SEL), jnp.float32),
    ],
    scratch_shapes=[
        pltpu.VMEM((B, R, 128), jnp.float32),
        pltpu.VMEM((B, G, 128), jnp.float32),
    ],
)

N_FEAT_ROWS = B * K_CROSS + B * K_SEL
NW = 32
F_PER_W = N_FEAT_ROWS // NW


@functools.partial(
    pl.kernel,
    out_type=jax.ShapeDtypeStruct((N_FEAT_ROWS, D), jnp.float32),
    mesh=plsc.VectorSubcoreMesh(core_axis_name="c", subcore_axis_name="s"),
    scratch_types=[
        pltpu.VMEM((2, 80), jnp.int32),
        pltpu.VMEM((F_PER_W, D), jnp.float32),
        pltpu.SemaphoreType.DMA,
    ],
)
def _sc_gather(feat_hbm, fidx_hbm, ofeat, fidx_v, frows_v, semf):
    wid = lax.axis_index("s") * 2 + lax.axis_index("c")
    pltpu.sync_copy(fidx_hbm.at[pl.ds(2 * wid, 2)], fidx_v)
    c1 = pltpu.async_copy(feat_hbm.at[fidx_v.at[0]], frows_v.at[pl.ds(0, 80)], semf)
    c2 = pltpu.async_copy(feat_hbm.at[fidx_v.at[1]], frows_v.at[pl.ds(80, 80)], semf)
    c1.wait()
    c2.wait()
    pltpu.sync_copy(frows_v, ofeat.at[pl.ds(F_PER_W * wid, F_PER_W)])


def kernel(centerness, cls_scores, points, features):
    cls_t = jnp.transpose(cls_scores, (0, 2, 1))
    cls_t = jnp.pad(cls_t, ((0, 0), (0, 0), (0, NPAD - N))).reshape(B, C, R, 128)
    cen = jnp.pad(centerness.reshape(B, N), ((0, 0), (0, NPAD - N))).reshape(B, R, 128)

    pts_t = jnp.pad(jnp.transpose(points, (0, 2, 1)),
                    ((0, 0), (0, 0), (0, NPAD - N))).reshape(B, 3, R, 128)

    cross_g, sort3, sel_pts_t = _tc_topk(cls_t, cen, pts_t)
    sort_inds = sort3.reshape(B, K_SEL)
    cross_g = cross_g.reshape(B, K_CROSS)

    offs = (jnp.arange(B, dtype=jnp.int32) * N).reshape(B, 1)
    sel_g = (sort_inds + offs).reshape(-1)
    fidx = jnp.concatenate([cross_g.reshape(-1), sel_g]).reshape(64, 80)

    feat_flat = features.reshape(B * N, D)
    gfeat = _sc_gather(feat_flat, fidx)

    cross_features = gfeat[: B * K_CROSS].reshape(B, K_CROSS, D)
    sel_features = gfeat[B * K_CROSS:].reshape(B, K_SEL, D)
    sel_points = jnp.transpose(sel_pts_t, (0, 2, 1))
    return (sel_points, sel_features, sort_inds, cross_features)
